# baseline (speedup 1.0000x reference)
import numpy as np
import jax
import jax.numpy as jnp
from jax.experimental import pallas as pl
from jax.experimental.pallas import tpu as pltpu

_T = 1000


def _make_tables():
    beta = np.linspace(0.0001, 0.02, _T, dtype=np.float32)
    alpha = (1.0 - beta).astype(np.float32)
    alpha_hat = np.cumprod(alpha, dtype=np.float32)
    sa = np.sqrt(alpha_hat).astype(np.float32)
    sb = np.sqrt((1.0 - alpha_hat).astype(np.float32)).astype(np.float32)
    return sa, sb


_SA, _SB = _make_tables()

_BM = 8


def _body(ts_ref, sa_ref, sb_ref, x_ref, e_ref, o_ref):
    base = pl.program_id(0) * _BM
    ca, cb = [], []
    for r in range(_BM):
        t = ts_ref[base + r]
        ca.append(sa_ref[t])
        cb.append(sb_ref[t])
    a = jnp.stack(ca).reshape(_BM, 1)
    b = jnp.stack(cb).reshape(_BM, 1)
    o_ref[...] = a * x_ref[...] + b * e_ref[...]


def kernel(x0, timesteps, eps):
    B = x0.shape[0]
    N = x0.shape[1] * x0.shape[2] * x0.shape[3]
    xf = x0.reshape(B, N)
    ef = eps.reshape(B, N)
    ts = timesteps.astype(jnp.int32)
    sa = jnp.asarray(_SA)
    sb = jnp.asarray(_SB)

    grid = (B // _BM,)
    smem = pl.BlockSpec(memory_space=pltpu.SMEM)
    blk = pl.BlockSpec((_BM, N), lambda i: (i, 0))
    out = pl.pallas_call(
        _body,
        grid=grid,
        in_specs=[smem, smem, smem, blk, blk],
        out_specs=blk,
        out_shape=jax.ShapeDtypeStruct((B, N), jnp.float32),
    )(ts, sa, sb, xf, ef)
    return (out.reshape(x0.shape), eps)


# speedup vs baseline: 2.9051x; 2.9051x over previous
import numpy as np
import jax
import jax.numpy as jnp
from jax.experimental import pallas as pl
from jax.experimental.pallas import tpu as pltpu

_T = 1000


def _make_tables():
    beta = np.linspace(0.0001, 0.02, _T, dtype=np.float32)
    alpha = (1.0 - beta).astype(np.float32)
    alpha_hat = np.cumprod(alpha, dtype=np.float32)
    sa = np.sqrt(alpha_hat).astype(np.float32)
    sb = np.sqrt((1.0 - alpha_hat).astype(np.float32)).astype(np.float32)
    return sa, sb


_SA, _SB = _make_tables()

_BM = 8


def _body(ts_ref, sa_ref, sb_ref, x_ref, e_ref, o_ref):
    base = pl.program_id(0) * _BM
    ca, cb = [], []
    for r in range(_BM):
        t = ts_ref[base + r]
        ca.append(sa_ref[t])
        cb.append(sb_ref[t])
    a = jnp.stack(ca).reshape(_BM, 1, 1, 1)
    b = jnp.stack(cb).reshape(_BM, 1, 1, 1)
    o_ref[...] = a * x_ref[...] + b * e_ref[...]


def kernel(x0, timesteps, eps):
    B, C, H, W = x0.shape
    ts = timesteps.astype(jnp.int32)
    sa = jnp.asarray(_SA)
    sb = jnp.asarray(_SB)

    grid = (B // _BM,)
    smem = pl.BlockSpec(memory_space=pltpu.SMEM)
    blk = pl.BlockSpec((_BM, C, H, W), lambda i: (i, 0, 0, 0))
    out = pl.pallas_call(
        _body,
        grid=grid,
        in_specs=[smem, smem, smem, blk, blk],
        out_specs=blk,
        out_shape=jax.ShapeDtypeStruct((B, C, H, W), jnp.float32),
    )(ts, sa, sb, x0, eps)
    return (out, eps)
